# baseline (speedup 1.0000x reference)
import jax
import jax.numpy as jnp
from jax import lax
from jax.experimental import pallas as pl
from jax.experimental.pallas import tpu as pltpu
from jax.experimental.pallas import tpu_sc as plsc

EMBED = 64
BATCH = 16384
NC = 2
NS = 16
NW = NC * NS
B_PER_W = BATCH // NW
CHUNK = 128
N_CHUNKS = B_PER_W // CHUNK
NVEC = B_PER_W // 16


def _sc_gather_dot(uidx_hbm, midx_hbm, uemb_hbm, memb_hbm, ubias_hbm,
                   mbias_hbm, partials_hbm, bsum_hbm,
                   uidx_v, midx_v, urows_v, mrows_v,
                   ub1_v, mb1_v, bsv, pacc_v, sem_e, sem_b):
    wid = lax.axis_index("s") * NC + lax.axis_index("c")
    base = wid * B_PER_W

    pltpu.sync_copy(uidx_hbm.at[wid], uidx_v)
    pltpu.sync_copy(midx_hbm.at[wid], midx_v)

    emb_copies = []
    for c in range(N_CHUNKS):
        rows = pl.ds(c * CHUNK, CHUNK)
        emb_copies.append(pltpu.async_copy(
            uemb_hbm.at[uidx_v.at[c]], urows_v.at[rows, :], sem_e))
        emb_copies.append(pltpu.async_copy(
            memb_hbm.at[midx_v.at[c]], mrows_v.at[rows, :], sem_e))

    bias_copies = []
    for c in range(N_CHUNKS):
        rows = pl.ds(c * CHUNK, CHUNK)
        bias_copies.append(pltpu.async_copy(
            ubias_hbm.at[uidx_v.at[c]], ub1_v.at[rows], sem_b))
        bias_copies.append(pltpu.async_copy(
            mbias_hbm.at[midx_v.at[c]], mb1_v.at[rows], sem_b))
    for cp in bias_copies:
        cp.wait()

    for k in range(B_PER_W // 16):
        s = pl.ds(k * 16, 16)
        bsv[s] = ub1_v[s] + mb1_v[s]
    pltpu.sync_copy(bsv, bsum_hbm.at[pl.ds(base, B_PER_W)])

    for cp in emb_copies:
        cp.wait()

    zero = jnp.zeros((16,), jnp.float32)

    def body(b, accs):
        a0, a1, a2, a3 = accs
        a0 = a0 + urows_v[b, pl.ds(0, 16)] * mrows_v[b, pl.ds(0, 16)]
        a1 = a1 + urows_v[b, pl.ds(16, 16)] * mrows_v[b, pl.ds(16, 16)]
        a2 = a2 + urows_v[b, pl.ds(32, 16)] * mrows_v[b, pl.ds(32, 16)]
        a3 = a3 + urows_v[b, pl.ds(48, 16)] * mrows_v[b, pl.ds(48, 16)]
        return (a0, a1, a2, a3)

    a0, a1, a2, a3 = lax.fori_loop(0, B_PER_W, body, (zero, zero, zero, zero))
    pacc_v[...] = (a0 + a1) + (a2 + a3)
    pltpu.sync_copy(pacc_v, partials_hbm.at[wid])


def _combine(p_ref, bs_ref, o_ref):
    s = jnp.sum(p_ref[...])
    o_ref[...] = jax.nn.sigmoid(s + bs_ref[...])


@jax.jit
def kernel(inputs, user_embedding, movie_embedding, user_bias, movie_bias):
    u_idx = inputs[:, 0].astype(jnp.int32).reshape(NW, N_CHUNKS, CHUNK)
    m_idx = inputs[:, 1].astype(jnp.int32).reshape(NW, N_CHUNKS, CHUNK)
    ub1 = user_bias.reshape(-1)
    mb1 = movie_bias.reshape(-1)

    mesh = plsc.VectorSubcoreMesh(
        core_axis_name="c", subcore_axis_name="s",
        num_cores=NC, num_subcores=NS)

    sc = pl.kernel(
        _sc_gather_dot,
        out_type=[
            jax.ShapeDtypeStruct((NW, 16), jnp.float32),
            jax.ShapeDtypeStruct((BATCH,), jnp.float32),
        ],
        mesh=mesh,
        scratch_types=[
            pltpu.VMEM((N_CHUNKS, CHUNK), jnp.int32),
            pltpu.VMEM((N_CHUNKS, CHUNK), jnp.int32),
            pltpu.VMEM((B_PER_W, EMBED), jnp.float32),
            pltpu.VMEM((B_PER_W, EMBED), jnp.float32),
            pltpu.VMEM((B_PER_W,), jnp.float32),
            pltpu.VMEM((B_PER_W,), jnp.float32),
            pltpu.VMEM((B_PER_W,), jnp.float32),
            pltpu.VMEM((16,), jnp.float32),
            pltpu.SemaphoreType.DMA,
            pltpu.SemaphoreType.DMA,
        ],
        compiler_params=pltpu.CompilerParams(use_tc_tiling_on_sc=False),
    )
    partials, bsum = sc(u_idx, m_idx, user_embedding, movie_embedding,
                        ub1, mb1)

    out = pl.pallas_call(
        _combine,
        out_shape=jax.ShapeDtypeStruct((128, 128), jnp.float32),
    )(partials, bsum.reshape(128, 128))
    return out.reshape(BATCH, 1)


# speedup vs baseline: 1.1106x; 1.1106x over previous
import jax
import jax.numpy as jnp
from jax import lax
from jax.experimental import pallas as pl
from jax.experimental.pallas import tpu as pltpu
from jax.experimental.pallas import tpu_sc as plsc

EMBED = 64
BATCH = 16384
NROWS = 1_000_000
NC = 2
NS = 16
NW = NC * NS
B_PER_W = BATCH // NW
CHUNK = 128
N_CHUNKS = B_PER_W // CHUNK

TW = 2048
TGRID = (NROWS + TW - 1) // TW
PU_ROWS = TGRID * (TW // 2)
PM_ROWS = NROWS // 2


def _transpose_pack(inT_ref, o_ref):
    x = inT_ref[...]
    xt = jnp.transpose(x)
    o_ref[...] = jnp.concatenate([xt[: TW // 2], xt[TW // 2:]], axis=1)


def _sc_bias(uidx_hbm, midx_hbm, ubias_hbm, mbias_hbm, bsum_hbm,
             uidx_v, midx_v, ub1_v, mb1_v, bsv, sem_b):
    wid = lax.axis_index("s") * NC + lax.axis_index("c")
    base = wid * B_PER_W

    pltpu.sync_copy(uidx_hbm.at[pl.ds(base, B_PER_W)], uidx_v)
    pltpu.sync_copy(midx_hbm.at[pl.ds(base, B_PER_W)], midx_v)

    bias_copies = []
    for c in range(N_CHUNKS):
        rows = pl.ds(c * CHUNK, CHUNK)
        bias_copies.append(pltpu.async_copy(
            ubias_hbm.at[uidx_v.at[rows]], ub1_v.at[rows], sem_b))
        bias_copies.append(pltpu.async_copy(
            mbias_hbm.at[midx_v.at[rows]], mb1_v.at[rows], sem_b))
    for cp in bias_copies:
        cp.wait()

    for k in range(B_PER_W // 16):
        s = pl.ds(k * 16, 16)
        bsv[s] = ub1_v[s] + mb1_v[s]
    pltpu.sync_copy(bsv, bsum_hbm.at[pl.ds(base, B_PER_W)])


def _sc_dot(uidx_hbm, midx_hbm, uP_hbm, mP_hbm, partials_hbm,
            uidx_v, midx_v, up_v, mp_v, uoff_v, moff_v,
            uslab0, mslab0, uslab1, mslab1, pacc_v,
            sem_u, sem_m):
    wid = lax.axis_index("s") * NC + lax.axis_index("c")
    base = wid * B_PER_W

    pltpu.sync_copy(uidx_hbm.at[pl.ds(base, B_PER_W)], uidx_v)
    pltpu.sync_copy(midx_hbm.at[pl.ds(base, B_PER_W)], midx_v)

    for k in range(B_PER_W // 16):
        s = pl.ds(k * 16, 16)
        ru = uidx_v[s]
        rm = midx_v[s]
        up_v[s] = lax.bitwise_or(
            lax.shift_left(lax.shift_right_logical(ru, 11), 10),
            lax.bitwise_and(ru, 1023))
        mp_v[s] = lax.bitwise_or(
            lax.shift_left(lax.shift_right_logical(rm, 11), 10),
            lax.bitwise_and(rm, 1023))
        uoff_v[s] = lax.bitwise_and(lax.shift_right_logical(ru, 10), 1) * 64
        moff_v[s] = lax.bitwise_and(lax.shift_right_logical(rm, 10), 1) * 64

    uslabs = (uslab0, uslab1)
    mslabs = (mslab0, mslab1)

    def fire(c):
        rows = pl.ds(c * CHUNK, CHUNK)
        cu = pltpu.async_copy(uP_hbm.at[up_v.at[rows]], uslabs[c % 2], sem_u)
        cm = pltpu.async_copy(mP_hbm.at[mp_v.at[rows]], mslabs[c % 2], sem_m)
        return cu, cm

    zero = jnp.zeros((16,), jnp.float32)
    accs = (zero, zero, zero, zero)
    pend = fire(0)
    for c in range(N_CHUNKS):
        nxt = fire(c + 1) if c + 1 < N_CHUNKS else None
        pend[0].wait()
        pend[1].wait()
        uslab = uslabs[c % 2]
        mslab = mslabs[c % 2]

        def body(g, a, c=c, uslab=uslab, mslab=mslab):
            a0, a1, a2, a3 = a
            uoff16 = uoff_v[pl.ds(c * CHUNK + g * 16, 16)]
            moff16 = moff_v[pl.ds(c * CHUNK + g * 16, 16)]
            for lane in range(16):
                j = g * 16 + lane
                off_u = uoff16[lane]
                off_m = moff16[lane]
                a0 = a0 + uslab[j, pl.ds(off_u, 16)] * mslab[j, pl.ds(off_m, 16)]
                a1 = a1 + uslab[j, pl.ds(off_u + 16, 16)] * mslab[j, pl.ds(off_m + 16, 16)]
                a2 = a2 + uslab[j, pl.ds(off_u + 32, 16)] * mslab[j, pl.ds(off_m + 32, 16)]
                a3 = a3 + uslab[j, pl.ds(off_u + 48, 16)] * mslab[j, pl.ds(off_m + 48, 16)]
            return (a0, a1, a2, a3)

        accs = lax.fori_loop(0, CHUNK // 16, body, accs)
        pend = nxt

    a0, a1, a2, a3 = accs
    pacc_v[...] = (a0 + a1) + (a2 + a3)
    pltpu.sync_copy(pacc_v, partials_hbm.at[pl.ds(wid * 16, 16)])


def _combine(p_ref, bs_ref, o_ref):
    s = jnp.sum(p_ref[...])
    o_ref[...] = jax.nn.sigmoid(s + bs_ref[...])


@jax.jit
def kernel(inputs, user_embedding, movie_embedding, user_bias, movie_bias):
    u_idx = inputs[:, 0].astype(jnp.int32)
    m_idx = inputs[:, 1].astype(jnp.int32)
    ub1 = user_bias.reshape(-1)
    mb1 = movie_bias.reshape(-1)
    tpose = pl.pallas_call(
        _transpose_pack,
        grid=(TGRID,),
        in_specs=[pl.BlockSpec((EMBED, TW), lambda i: (0, i))],
        out_specs=pl.BlockSpec((TW // 2, 128), lambda i: (i, 0)),
        out_shape=jax.ShapeDtypeStruct((PU_ROWS, 128), jnp.float32),
    )
    uP = tpose(user_embedding.T)
    mP = tpose(movie_embedding.T)

    mesh = plsc.VectorSubcoreMesh(
        core_axis_name="c", subcore_axis_name="s",
        num_cores=NC, num_subcores=NS)

    bsum = pl.kernel(
        _sc_bias,
        out_type=jax.ShapeDtypeStruct((BATCH,), jnp.float32),
        mesh=mesh,
        scratch_types=[
            pltpu.VMEM((B_PER_W,), jnp.int32),
            pltpu.VMEM((B_PER_W,), jnp.int32),
            pltpu.VMEM((B_PER_W,), jnp.float32),
            pltpu.VMEM((B_PER_W,), jnp.float32),
            pltpu.VMEM((B_PER_W,), jnp.float32),
            pltpu.SemaphoreType.DMA,
        ],
        compiler_params=pltpu.CompilerParams(use_tc_tiling_on_sc=False),
    )(u_idx, m_idx, ub1, mb1)

    partials = pl.kernel(
        _sc_dot,
        out_type=jax.ShapeDtypeStruct((NW * 16,), jnp.float32),
        mesh=mesh,
        scratch_types=[
            pltpu.VMEM((B_PER_W,), jnp.int32),
            pltpu.VMEM((B_PER_W,), jnp.int32),
            pltpu.VMEM((B_PER_W,), jnp.int32),
            pltpu.VMEM((B_PER_W,), jnp.int32),
            pltpu.VMEM((B_PER_W,), jnp.int32),
            pltpu.VMEM((B_PER_W,), jnp.int32),
            pltpu.VMEM((CHUNK, 128), jnp.float32),
            pltpu.VMEM((CHUNK, 128), jnp.float32),
            pltpu.VMEM((CHUNK, 128), jnp.float32),
            pltpu.VMEM((CHUNK, 128), jnp.float32),
            pltpu.VMEM((16,), jnp.float32),
            pltpu.SemaphoreType.DMA,
            pltpu.SemaphoreType.DMA,
        ],
        compiler_params=pltpu.CompilerParams(use_tc_tiling_on_sc=True),
    )(u_idx, m_idx, uP, mP)

    out = pl.pallas_call(
        _combine,
        out_shape=jax.ShapeDtypeStruct((128, 128), jnp.float32),
    )(partials.reshape(4, 128), bsum.reshape(128, 128))
    return out.reshape(BATCH, 1)


# speedup vs baseline: 1.4629x; 1.3173x over previous
import jax
import jax.numpy as jnp
from jax import lax
from jax.experimental import pallas as pl
from jax.experimental.pallas import tpu as pltpu
from jax.experimental.pallas import tpu_sc as plsc

EMBED = 64
BATCH = 16384
NROWS = 1_000_000
NC = 2
NS = 16
NW = NC * NS
B_PER_W = BATCH // NW
CHUNK = 128
N_CHUNKS = B_PER_W // CHUNK

TW = 4096
TGRID = (NROWS + TW - 1) // TW
PU_ROWS = TGRID * (TW // 2)
HBITS = 11
QMASK = TW // 2 - 1


def _transpose_pack(inT_ref, o_ref):
    x = inT_ref[...]
    eye = (lax.broadcasted_iota(jnp.int32, (EMBED, EMBED), 0)
           == lax.broadcasted_iota(jnp.int32, (EMBED, EMBED), 1)
           ).astype(jnp.float32)
    dn = (((0,), (0,)), ((), ()))
    o_ref[:, 0:EMBED] = lax.dot_general(
        x[:, : TW // 2], eye, dn, preferred_element_type=jnp.float32)
    o_ref[:, EMBED:128] = lax.dot_general(
        x[:, TW // 2:], eye, dn, preferred_element_type=jnp.float32)


def _sc_bias(uidx_hbm, midx_hbm, ubias_hbm, mbias_hbm, bsum_hbm,
             uidx_v, midx_v, ub1_v, mb1_v, bsv, sem_b):
    wid = lax.axis_index("s") * NC + lax.axis_index("c")
    base = wid * B_PER_W

    pltpu.sync_copy(uidx_hbm.at[pl.ds(base, B_PER_W)], uidx_v)
    pltpu.sync_copy(midx_hbm.at[pl.ds(base, B_PER_W)], midx_v)

    bias_copies = []
    for c in range(N_CHUNKS):
        rows = pl.ds(c * CHUNK, CHUNK)
        bias_copies.append(pltpu.async_copy(
            ubias_hbm.at[uidx_v.at[rows]], ub1_v.at[rows], sem_b))
        bias_copies.append(pltpu.async_copy(
            mbias_hbm.at[midx_v.at[rows]], mb1_v.at[rows], sem_b))
    for cp in bias_copies:
        cp.wait()

    for k in range(B_PER_W // 16):
        s = pl.ds(k * 16, 16)
        bsv[s] = ub1_v[s] + mb1_v[s]
    pltpu.sync_copy(bsv, bsum_hbm.at[pl.ds(base, B_PER_W)])


def _sc_dot(uidx_hbm, midx_hbm, uP_hbm, mP_hbm, partials_hbm,
            uidx_v, midx_v, up_v, mp_v, uoff_v, moff_v,
            uslab0, mslab0, uslab1, mslab1, pacc_v,
            sem_u, sem_m):
    wid = lax.axis_index("s") * NC + lax.axis_index("c")
    base = wid * B_PER_W

    pltpu.sync_copy(uidx_hbm.at[pl.ds(base, B_PER_W)], uidx_v)
    pltpu.sync_copy(midx_hbm.at[pl.ds(base, B_PER_W)], midx_v)

    for k in range(B_PER_W // 16):
        s = pl.ds(k * 16, 16)
        ru = uidx_v[s]
        rm = midx_v[s]
        up_v[s] = lax.bitwise_or(
            lax.shift_left(lax.shift_right_logical(ru, HBITS + 1), HBITS),
            lax.bitwise_and(ru, QMASK))
        mp_v[s] = lax.bitwise_or(
            lax.shift_left(lax.shift_right_logical(rm, HBITS + 1), HBITS),
            lax.bitwise_and(rm, QMASK))
        uoff_v[s] = lax.bitwise_and(
            lax.shift_right_logical(ru, HBITS), 1) * EMBED
        moff_v[s] = lax.bitwise_and(
            lax.shift_right_logical(rm, HBITS), 1) * EMBED

    uslabs = (uslab0, uslab1)
    mslabs = (mslab0, mslab1)

    def fire(c):
        rows = pl.ds(c * CHUNK, CHUNK)
        cu = pltpu.async_copy(uP_hbm.at[up_v.at[rows]], uslabs[c % 2], sem_u)
        cm = pltpu.async_copy(mP_hbm.at[mp_v.at[rows]], mslabs[c % 2], sem_m)
        return cu, cm

    zero = jnp.zeros((16,), jnp.float32)
    accs = (zero, zero, zero, zero)
    pend = fire(0)
    for c in range(N_CHUNKS):
        nxt = fire(c + 1) if c + 1 < N_CHUNKS else None
        pend[0].wait()
        pend[1].wait()
        uslab = uslabs[c % 2]
        mslab = mslabs[c % 2]

        def body(g, a, c=c, uslab=uslab, mslab=mslab):
            a0, a1, a2, a3 = a
            uoff16 = uoff_v[pl.ds(c * CHUNK + g * 16, 16)]
            moff16 = moff_v[pl.ds(c * CHUNK + g * 16, 16)]
            for lane in range(16):
                j = g * 16 + lane
                off_u = uoff16[lane]
                off_m = moff16[lane]
                a0 = a0 + uslab[j, pl.ds(off_u, 16)] * mslab[j, pl.ds(off_m, 16)]
                a1 = a1 + uslab[j, pl.ds(off_u + 16, 16)] * mslab[j, pl.ds(off_m + 16, 16)]
                a2 = a2 + uslab[j, pl.ds(off_u + 32, 16)] * mslab[j, pl.ds(off_m + 32, 16)]
                a3 = a3 + uslab[j, pl.ds(off_u + 48, 16)] * mslab[j, pl.ds(off_m + 48, 16)]
            return (a0, a1, a2, a3)

        accs = lax.fori_loop(0, CHUNK // 16, body, accs)
        pend = nxt

    a0, a1, a2, a3 = accs
    pacc_v[...] = (a0 + a1) + (a2 + a3)
    pltpu.sync_copy(pacc_v, partials_hbm.at[pl.ds(wid * 16, 16)])


def _combine(p_ref, bs_ref, o_ref):
    s = jnp.sum(p_ref[...])
    o_ref[...] = jax.nn.sigmoid(s + bs_ref[...])


@jax.jit
def kernel(inputs, user_embedding, movie_embedding, user_bias, movie_bias):
    u_idx = inputs[:, 0].astype(jnp.int32)
    m_idx = inputs[:, 1].astype(jnp.int32)
    ub1 = user_bias.reshape(-1)
    mb1 = movie_bias.reshape(-1)
    tpose = pl.pallas_call(
        _transpose_pack,
        grid=(TGRID,),
        in_specs=[pl.BlockSpec((EMBED, TW), lambda i: (0, i))],
        out_specs=pl.BlockSpec((TW // 2, 128), lambda i: (i, 0)),
        out_shape=jax.ShapeDtypeStruct((PU_ROWS, 128), jnp.float32),
    )
    uP = tpose(user_embedding.T)
    mP = tpose(movie_embedding.T)

    mesh = plsc.VectorSubcoreMesh(
        core_axis_name="c", subcore_axis_name="s",
        num_cores=NC, num_subcores=NS)

    bsum = pl.kernel(
        _sc_bias,
        out_type=jax.ShapeDtypeStruct((BATCH,), jnp.float32),
        mesh=mesh,
        scratch_types=[
            pltpu.VMEM((B_PER_W,), jnp.int32),
            pltpu.VMEM((B_PER_W,), jnp.int32),
            pltpu.VMEM((B_PER_W,), jnp.float32),
            pltpu.VMEM((B_PER_W,), jnp.float32),
            pltpu.VMEM((B_PER_W,), jnp.float32),
            pltpu.SemaphoreType.DMA,
        ],
        compiler_params=pltpu.CompilerParams(use_tc_tiling_on_sc=False),
    )(u_idx, m_idx, ub1, mb1)

    partials = pl.kernel(
        _sc_dot,
        out_type=jax.ShapeDtypeStruct((NW * 16,), jnp.float32),
        mesh=mesh,
        scratch_types=[
            pltpu.VMEM((B_PER_W,), jnp.int32),
            pltpu.VMEM((B_PER_W,), jnp.int32),
            pltpu.VMEM((B_PER_W,), jnp.int32),
            pltpu.VMEM((B_PER_W,), jnp.int32),
            pltpu.VMEM((B_PER_W,), jnp.int32),
            pltpu.VMEM((B_PER_W,), jnp.int32),
            pltpu.VMEM((CHUNK, 128), jnp.float32),
            pltpu.VMEM((CHUNK, 128), jnp.float32),
            pltpu.VMEM((CHUNK, 128), jnp.float32),
            pltpu.VMEM((CHUNK, 128), jnp.float32),
            pltpu.VMEM((16,), jnp.float32),
            pltpu.SemaphoreType.DMA,
            pltpu.SemaphoreType.DMA,
        ],
        compiler_params=pltpu.CompilerParams(use_tc_tiling_on_sc=True),
    )(u_idx, m_idx, uP, mP)

    out = pl.pallas_call(
        _combine,
        out_shape=jax.ShapeDtypeStruct((128, 128), jnp.float32),
    )(partials.reshape(4, 128), bsum.reshape(128, 128))
    return out.reshape(BATCH, 1)


# speedup vs baseline: 1.9936x; 1.3628x over previous
import jax
import jax.numpy as jnp
from jax import lax
from jax.experimental import pallas as pl
from jax.experimental.pallas import tpu as pltpu
from jax.experimental.pallas import tpu_sc as plsc

EMBED = 64
BATCH = 16384
NROWS = 1_000_000
NC = 2
NS = 16
NW = NC * NS
B_PER_W = BATCH // NW
CHUNK = 128
N_CHUNKS = B_PER_W // CHUNK

TW = 16384
TGRID = (NROWS + TW - 1) // TW
PU_ROWS = TGRID * (TW // 2)
HBITS = 13
QMASK = TW // 2 - 1


def _transpose_pack(inT_ref, o_ref):
    x = inT_ref[...]
    eye = (lax.broadcasted_iota(jnp.int32, (EMBED, EMBED), 0)
           == lax.broadcasted_iota(jnp.int32, (EMBED, EMBED), 1)
           ).astype(jnp.float32)
    dn = (((0,), (0,)), ((), ()))
    o_ref[:, 0:EMBED] = lax.dot_general(
        x[:, : TW // 2], eye, dn, preferred_element_type=jnp.float32)
    o_ref[:, EMBED:128] = lax.dot_general(
        x[:, TW // 2:], eye, dn, preferred_element_type=jnp.float32)


def _sc_bias(uidx_hbm, midx_hbm, ubias_hbm, mbias_hbm, bsum_hbm,
             uidx_v, midx_v, ub1_v, mb1_v, bsv, sem_b):
    wid = lax.axis_index("s") * NC + lax.axis_index("c")
    base = wid * B_PER_W

    pltpu.sync_copy(uidx_hbm.at[pl.ds(base, B_PER_W)], uidx_v)
    pltpu.sync_copy(midx_hbm.at[pl.ds(base, B_PER_W)], midx_v)

    bias_copies = []
    for c in range(N_CHUNKS):
        rows = pl.ds(c * CHUNK, CHUNK)
        bias_copies.append(pltpu.async_copy(
            ubias_hbm.at[uidx_v.at[rows]], ub1_v.at[rows], sem_b))
        bias_copies.append(pltpu.async_copy(
            mbias_hbm.at[midx_v.at[rows]], mb1_v.at[rows], sem_b))
    for cp in bias_copies:
        cp.wait()

    for k in range(B_PER_W // 16):
        s = pl.ds(k * 16, 16)
        bsv[s] = ub1_v[s] + mb1_v[s]
    pltpu.sync_copy(bsv, bsum_hbm.at[pl.ds(base, B_PER_W)])


def _sc_dot(uidx_hbm, midx_hbm, uP_hbm, mP_hbm, partials_hbm,
            uidx_v, midx_v, up_v, mp_v, uoff_v, moff_v,
            uslab0, mslab0, uslab1, mslab1, pacc_v,
            sem_u, sem_m):
    wid = lax.axis_index("s") * NC + lax.axis_index("c")
    base = wid * B_PER_W

    pltpu.sync_copy(uidx_hbm.at[pl.ds(base, B_PER_W)], uidx_v)
    pltpu.sync_copy(midx_hbm.at[pl.ds(base, B_PER_W)], midx_v)

    for k in range(B_PER_W // 16):
        s = pl.ds(k * 16, 16)
        ru = uidx_v[s]
        rm = midx_v[s]
        up_v[s] = lax.bitwise_or(
            lax.shift_left(lax.shift_right_logical(ru, HBITS + 1), HBITS),
            lax.bitwise_and(ru, QMASK))
        mp_v[s] = lax.bitwise_or(
            lax.shift_left(lax.shift_right_logical(rm, HBITS + 1), HBITS),
            lax.bitwise_and(rm, QMASK))
        uoff_v[s] = lax.bitwise_and(
            lax.shift_right_logical(ru, HBITS), 1) * EMBED
        moff_v[s] = lax.bitwise_and(
            lax.shift_right_logical(rm, HBITS), 1) * EMBED

    uslabs = (uslab0, uslab1)
    mslabs = (mslab0, mslab1)

    def fire(c):
        rows = pl.ds(c * CHUNK, CHUNK)
        cu = pltpu.async_copy(uP_hbm.at[up_v.at[rows]], uslabs[c % 2], sem_u)
        cm = pltpu.async_copy(mP_hbm.at[mp_v.at[rows]], mslabs[c % 2], sem_m)
        return cu, cm

    zero = jnp.zeros((16,), jnp.float32)
    accs = (zero, zero, zero, zero)
    pend = fire(0)
    for c in range(N_CHUNKS):
        nxt = fire(c + 1) if c + 1 < N_CHUNKS else None
        pend[0].wait()
        pend[1].wait()
        uslab = uslabs[c % 2]
        mslab = mslabs[c % 2]

        def body(g, a, c=c, uslab=uslab, mslab=mslab):
            a0, a1, a2, a3 = a
            uoff16 = uoff_v[pl.ds(c * CHUNK + g * 16, 16)]
            moff16 = moff_v[pl.ds(c * CHUNK + g * 16, 16)]
            for lane in range(16):
                j = g * 16 + lane
                off_u = uoff16[lane]
                off_m = moff16[lane]
                a0 = a0 + uslab[j, pl.ds(off_u, 16)] * mslab[j, pl.ds(off_m, 16)]
                a1 = a1 + uslab[j, pl.ds(off_u + 16, 16)] * mslab[j, pl.ds(off_m + 16, 16)]
                a2 = a2 + uslab[j, pl.ds(off_u + 32, 16)] * mslab[j, pl.ds(off_m + 32, 16)]
                a3 = a3 + uslab[j, pl.ds(off_u + 48, 16)] * mslab[j, pl.ds(off_m + 48, 16)]
            return (a0, a1, a2, a3)

        accs = lax.fori_loop(0, CHUNK // 16, body, accs)
        pend = nxt

    a0, a1, a2, a3 = accs
    pacc_v[...] = (a0 + a1) + (a2 + a3)
    pltpu.sync_copy(pacc_v, partials_hbm.at[pl.ds(wid * 16, 16)])


def _combine(p_ref, bs_ref, o_ref):
    s = jnp.sum(p_ref[...])
    o_ref[...] = jax.nn.sigmoid(s + bs_ref[...])


@jax.jit
def kernel(inputs, user_embedding, movie_embedding, user_bias, movie_bias):
    u_idx = inputs[:, 0].astype(jnp.int32)
    m_idx = inputs[:, 1].astype(jnp.int32)
    ub1 = user_bias.reshape(-1)
    mb1 = movie_bias.reshape(-1)
    tpose = pl.pallas_call(
        _transpose_pack,
        grid=(TGRID,),
        in_specs=[pl.BlockSpec((EMBED, TW), lambda i: (0, i))],
        out_specs=pl.BlockSpec((TW // 2, 128), lambda i: (i, 0)),
        out_shape=jax.ShapeDtypeStruct((PU_ROWS, 128), jnp.float32),
    )
    uP = tpose(user_embedding.T)
    mP = tpose(movie_embedding.T)

    mesh = plsc.VectorSubcoreMesh(
        core_axis_name="c", subcore_axis_name="s",
        num_cores=NC, num_subcores=NS)

    bsum = pl.kernel(
        _sc_bias,
        out_type=jax.ShapeDtypeStruct((BATCH,), jnp.float32),
        mesh=mesh,
        scratch_types=[
            pltpu.VMEM((B_PER_W,), jnp.int32),
            pltpu.VMEM((B_PER_W,), jnp.int32),
            pltpu.VMEM((B_PER_W,), jnp.float32),
            pltpu.VMEM((B_PER_W,), jnp.float32),
            pltpu.VMEM((B_PER_W,), jnp.float32),
            pltpu.SemaphoreType.DMA,
        ],
        compiler_params=pltpu.CompilerParams(use_tc_tiling_on_sc=False),
    )(u_idx, m_idx, ub1, mb1)

    partials = pl.kernel(
        _sc_dot,
        out_type=jax.ShapeDtypeStruct((NW * 16,), jnp.float32),
        mesh=mesh,
        scratch_types=[
            pltpu.VMEM((B_PER_W,), jnp.int32),
            pltpu.VMEM((B_PER_W,), jnp.int32),
            pltpu.VMEM((B_PER_W,), jnp.int32),
            pltpu.VMEM((B_PER_W,), jnp.int32),
            pltpu.VMEM((B_PER_W,), jnp.int32),
            pltpu.VMEM((B_PER_W,), jnp.int32),
            pltpu.VMEM((CHUNK, 128), jnp.float32),
            pltpu.VMEM((CHUNK, 128), jnp.float32),
            pltpu.VMEM((CHUNK, 128), jnp.float32),
            pltpu.VMEM((CHUNK, 128), jnp.float32),
            pltpu.VMEM((16,), jnp.float32),
            pltpu.SemaphoreType.DMA,
            pltpu.SemaphoreType.DMA,
        ],
        compiler_params=pltpu.CompilerParams(use_tc_tiling_on_sc=True),
    )(u_idx, m_idx, uP, mP)

    out = pl.pallas_call(
        _combine,
        out_shape=jax.ShapeDtypeStruct((128, 128), jnp.float32),
    )(partials.reshape(4, 128), bsum.reshape(128, 128))
    return out.reshape(BATCH, 1)


# speedup vs baseline: 2.5742x; 1.2912x over previous
import jax
import jax.numpy as jnp
from jax import lax
from jax.experimental import pallas as pl
from jax.experimental.pallas import tpu as pltpu
from jax.experimental.pallas import tpu_sc as plsc

EMBED = 64
BATCH = 16384
NROWS = 1_000_000
NC = 2
NS = 16
NW = NC * NS
B_PER_W = BATCH // NW
CHUNK = 128
N_CHUNKS = B_PER_W // CHUNK

TW = 16384
TGRID = (NROWS + TW - 1) // TW
PU_ROWS = TGRID * (TW // 2)
HBITS = 13
QMASK = TW // 2 - 1


def _transpose_pack(inT_ref, o_ref):
    x = inT_ref[...]
    eye = (lax.broadcasted_iota(jnp.int32, (EMBED, EMBED), 0)
           == lax.broadcasted_iota(jnp.int32, (EMBED, EMBED), 1)
           ).astype(jnp.float32)
    dn = (((0,), (0,)), ((), ()))
    o_ref[:, 0:EMBED] = lax.dot_general(
        x[:, : TW // 2], eye, dn, preferred_element_type=jnp.float32)
    o_ref[:, EMBED:128] = lax.dot_general(
        x[:, TW // 2:], eye, dn, preferred_element_type=jnp.float32)


def _sc_bias(uidx_hbm, midx_hbm, ubias_hbm, mbias_hbm, bsum_hbm,
             uidx_v, midx_v, ub1_v, mb1_v, bsv, sem_b):
    wid = lax.axis_index("s") * NC + lax.axis_index("c")
    base = wid * B_PER_W

    pltpu.sync_copy(uidx_hbm.at[pl.ds(base, B_PER_W)], uidx_v)
    pltpu.sync_copy(midx_hbm.at[pl.ds(base, B_PER_W)], midx_v)

    bias_copies = []
    for c in range(N_CHUNKS):
        rows = pl.ds(c * CHUNK, CHUNK)
        bias_copies.append(pltpu.async_copy(
            ubias_hbm.at[uidx_v.at[rows]], ub1_v.at[rows], sem_b))
        bias_copies.append(pltpu.async_copy(
            mbias_hbm.at[midx_v.at[rows]], mb1_v.at[rows], sem_b))
    for cp in bias_copies:
        cp.wait()

    for k in range(B_PER_W // 16):
        s = pl.ds(k * 16, 16)
        bsv[s] = ub1_v[s] + mb1_v[s]
    pltpu.sync_copy(bsv, bsum_hbm.at[pl.ds(base, B_PER_W)])


MCHUNK = 32
N_MCHUNKS = B_PER_W // MCHUNK


def _sc_dot(uidx_hbm, midx_hbm, uP_hbm, mP3_hbm, partials_hbm,
            uidx_v, midx_v, up_v, mp_v, uoff_v, moff_v,
            uslab0, mslab0, uslab1, mslab1, pacc_v,
            sem_u, sem_m):
    wid = lax.axis_index("s") * NC + lax.axis_index("c")
    base = wid * B_PER_W

    pltpu.sync_copy(uidx_hbm.at[pl.ds(base, B_PER_W)], uidx_v)
    pltpu.sync_copy(midx_hbm.at[pl.ds(base, B_PER_W)], midx_v)

    for k in range(B_PER_W // 16):
        s = pl.ds(k * 16, 16)
        ru = uidx_v[s]
        rm = midx_v[s]
        up_v[s] = lax.bitwise_or(
            lax.shift_left(lax.shift_right_logical(ru, HBITS + 1), HBITS),
            lax.bitwise_and(ru, QMASK))
        mp_v[s] = lax.shift_right_logical(rm, 3)
        uoff_v[s] = lax.bitwise_and(
            lax.shift_right_logical(ru, HBITS), 1) * EMBED
        moff_v[s] = lax.bitwise_and(rm, 7)

    uslabs = (uslab0, uslab1)
    mslabs = (mslab0, mslab1)

    def fire(c):
        urows = pl.ds(c * MCHUNK, MCHUNK)
        cu = pltpu.async_copy(uP_hbm.at[up_v.at[urows]], uslabs[c % 2], sem_u)

        def mgrp(g, _, c=c):
            v = mp_v[pl.ds(c * MCHUNK + g * 16, 16)]
            for lane in range(16):
                pltpu.async_copy(mP3_hbm.at[v[lane]],
                                 mslabs[c % 2].at[g * 16 + lane], sem_m)
            return 0

        lax.fori_loop(0, MCHUNK // 16, mgrp, 0)
        return cu

    def mdrain(c):
        pltpu.make_async_copy(mP3_hbm.at[pl.ds(0, MCHUNK)], mslabs[c % 2],
                              sem_m).wait()

    zero = jnp.zeros((16,), jnp.float32)
    accs = (zero, zero, zero, zero)
    pend = fire(0)
    for c in range(N_MCHUNKS):
        nxt = fire(c + 1) if c + 1 < N_MCHUNKS else None
        pend.wait()
        mdrain(c)
        uslab = uslabs[c % 2]
        mslab = mslabs[c % 2]

        def body(g, a, c=c, uslab=uslab, mslab=mslab):
            a0, a1, a2, a3 = a
            uoff16 = uoff_v[pl.ds(c * MCHUNK + g * 16, 16)]
            moff16 = moff_v[pl.ds(c * MCHUNK + g * 16, 16)]
            for lane in range(16):
                j = g * 16 + lane
                off_u = uoff16[lane]
                sub_m = moff16[lane]
                a0 = a0 + uslab[j, pl.ds(off_u, 16)] * mslab[j, sub_m, pl.ds(0, 16)]
                a1 = a1 + uslab[j, pl.ds(off_u + 16, 16)] * mslab[j, sub_m, pl.ds(16, 16)]
                a2 = a2 + uslab[j, pl.ds(off_u + 32, 16)] * mslab[j, sub_m, pl.ds(32, 16)]
                a3 = a3 + uslab[j, pl.ds(off_u + 48, 16)] * mslab[j, sub_m, pl.ds(48, 16)]
            return (a0, a1, a2, a3)

        accs = lax.fori_loop(0, MCHUNK // 16, body, accs)
        pend = nxt
        del nxt

    a0, a1, a2, a3 = accs
    pacc_v[...] = (a0 + a1) + (a2 + a3)
    pltpu.sync_copy(pacc_v, partials_hbm.at[pl.ds(wid * 16, 16)])


def _combine(p_ref, bs_ref, o_ref):
    s = jnp.sum(p_ref[...])
    o_ref[...] = jax.nn.sigmoid(s + bs_ref[...])


@jax.jit
def kernel(inputs, user_embedding, movie_embedding, user_bias, movie_bias):
    u_idx = inputs[:, 0].astype(jnp.int32)
    m_idx = inputs[:, 1].astype(jnp.int32)
    ub1 = user_bias.reshape(-1)
    mb1 = movie_bias.reshape(-1)
    tpose = pl.pallas_call(
        _transpose_pack,
        grid=(TGRID,),
        in_specs=[pl.BlockSpec((EMBED, TW), lambda i: (0, i))],
        out_specs=pl.BlockSpec((TW // 2, 128), lambda i: (i, 0)),
        out_shape=jax.ShapeDtypeStruct((PU_ROWS, 128), jnp.float32),
    )
    uP = tpose(user_embedding.T)
    mP3 = movie_embedding.reshape(NROWS // 8, 8, EMBED)

    mesh = plsc.VectorSubcoreMesh(
        core_axis_name="c", subcore_axis_name="s",
        num_cores=NC, num_subcores=NS)

    bsum = pl.kernel(
        _sc_bias,
        out_type=jax.ShapeDtypeStruct((BATCH,), jnp.float32),
        mesh=mesh,
        scratch_types=[
            pltpu.VMEM((B_PER_W,), jnp.int32),
            pltpu.VMEM((B_PER_W,), jnp.int32),
            pltpu.VMEM((B_PER_W,), jnp.float32),
            pltpu.VMEM((B_PER_W,), jnp.float32),
            pltpu.VMEM((B_PER_W,), jnp.float32),
            pltpu.SemaphoreType.DMA,
        ],
        compiler_params=pltpu.CompilerParams(use_tc_tiling_on_sc=False),
    )(u_idx, m_idx, ub1, mb1)

    partials = pl.kernel(
        _sc_dot,
        out_type=jax.ShapeDtypeStruct((NW * 16,), jnp.float32),
        mesh=mesh,
        scratch_types=[
            pltpu.VMEM((B_PER_W,), jnp.int32),
            pltpu.VMEM((B_PER_W,), jnp.int32),
            pltpu.VMEM((B_PER_W,), jnp.int32),
            pltpu.VMEM((B_PER_W,), jnp.int32),
            pltpu.VMEM((B_PER_W,), jnp.int32),
            pltpu.VMEM((B_PER_W,), jnp.int32),
            pltpu.VMEM((MCHUNK, 128), jnp.float32),
            pltpu.VMEM((MCHUNK, 8, EMBED), jnp.float32),
            pltpu.VMEM((MCHUNK, 128), jnp.float32),
            pltpu.VMEM((MCHUNK, 8, EMBED), jnp.float32),
            pltpu.VMEM((16,), jnp.float32),
            pltpu.SemaphoreType.DMA,
            pltpu.SemaphoreType.DMA,
        ],
        compiler_params=pltpu.CompilerParams(use_tc_tiling_on_sc=True),
    )(u_idx, m_idx, uP, mP3)

    out = pl.pallas_call(
        _combine,
        out_shape=jax.ShapeDtypeStruct((128, 128), jnp.float32),
    )(partials.reshape(4, 128), bsum.reshape(128, 128))
    return out.reshape(BATCH, 1)
